# baseline (speedup 1.0000x reference)
import functools

import jax
import jax.numpy as jnp
from jax import lax
from jax.experimental import pallas as pl
from jax.experimental.pallas import tpu as pltpu
from jax.experimental.pallas import tpu_sc as plsc

F0, F1, F2 = 5, 6, 2
EMB = 128
E = 320000
NROWS = F0 * F1 * F2

NC, NS = 2, 16
NW = NC * NS
PER_W = E // NW
CHUNK = 80
NCH = PER_W // CHUNK
NBUF = 5


def _table_body(w0_ref, w1_ref, w2_ref, c_ref):
    r = lax.broadcasted_iota(jnp.int32, (NROWS, 1), 0)
    i0 = r // (F1 * F2)
    i1 = (r // F2) % F1
    i2 = r % F2
    oh0 = (i0 == lax.broadcasted_iota(jnp.int32, (NROWS, F0), 1)).astype(jnp.float32)
    oh1 = (i1 == lax.broadcasted_iota(jnp.int32, (NROWS, F1), 1)).astype(jnp.float32)
    oh2 = (i2 == lax.broadcasted_iota(jnp.int32, (NROWS, F2), 1)).astype(jnp.float32)
    acc = jnp.dot(oh0, w0_ref[...], preferred_element_type=jnp.float32)
    acc = acc + jnp.dot(oh1, w1_ref[...], preferred_element_type=jnp.float32)
    acc = acc + jnp.dot(oh2, w2_ref[...], preferred_element_type=jnp.float32)
    c_ref[...] = acc


def _build_table(w0, w1, w2):
    return pl.pallas_call(
        _table_body,
        out_shape=jax.ShapeDtypeStruct((NROWS, EMB), jnp.float32),
    )(w0, w1, w2)


def _sc_body(a0_hbm, a1_hbm, a2_hbm, c_hbm, out_hbm,
             a0_v, a1_v, a2_v, idx_v, rows_v,
             isem, gsem, osem):
    wid = lax.axis_index("s") * NC + lax.axis_index("c")
    wbase = wid * PER_W

    def fire_in(k, b):
        base = wbase + k * CHUNK
        pltpu.async_copy(a0_hbm.at[pl.ds(base, CHUNK)], a0_v.at[b], isem.at[b])
        pltpu.async_copy(a1_hbm.at[pl.ds(base, CHUNK)], a1_v.at[b], isem.at[b])
        pltpu.async_copy(a2_hbm.at[pl.ds(base, CHUNK)], a2_v.at[b], isem.at[b])

    def wait_in(k, b):
        base = wbase + k * CHUNK
        pltpu.make_async_copy(a0_hbm.at[pl.ds(base, CHUNK)], a0_v.at[b], isem.at[b]).wait()
        pltpu.make_async_copy(a1_hbm.at[pl.ds(base, CHUNK)], a1_v.at[b], isem.at[b]).wait()
        pltpu.make_async_copy(a2_hbm.at[pl.ds(base, CHUNK)], a2_v.at[b], isem.at[b]).wait()

    def wait_out(k, b):
        base = wbase + k * CHUNK
        pltpu.make_async_copy(rows_v.at[b], out_hbm.at[pl.ds(base, CHUNK)],
                              osem.at[b]).wait()

    def step(k, b, fire_next, do_wait_out):
        wait_in(k, b)
        for i in range(CHUNK // 16):
            s = pl.ds(i * 16, 16)
            idx_v[b, s] = a0_v[b, s] * (F1 * F2) + a1_v[b, s] * F2 + a2_v[b, s]
        if fire_next:
            fire_in(k + NBUF, b)
        if do_wait_out:
            wait_out(k - NBUF, b)
        pltpu.async_copy(c_hbm.at[idx_v.at[b]], rows_v.at[b], gsem.at[b]).wait()
        base = wbase + k * CHUNK
        pltpu.async_copy(rows_v.at[b], out_hbm.at[pl.ds(base, CHUNK)], osem.at[b])

    for b in range(NBUF):
        fire_in(b, b)
    for b in range(NBUF):
        step(b, b, fire_next=True, do_wait_out=False)

    def super_step(g, carry):
        for b in range(NBUF):
            step(g * NBUF + b, b, fire_next=True, do_wait_out=True)
        return carry

    lax.fori_loop(1, NCH // NBUF - 1, super_step, 0)
    for b in range(NBUF):
        step((NCH - NBUF) + b, b, fire_next=False, do_wait_out=True)
    for b in range(NBUF):
        wait_out((NCH - NBUF) + b, b)


@functools.partial(jax.jit, static_argnames=())
def _sc_lookup(a0, a1, a2, table):
    mesh = plsc.VectorSubcoreMesh(core_axis_name="c", subcore_axis_name="s")
    fn = pl.kernel(
        _sc_body,
        out_type=jax.ShapeDtypeStruct((E, EMB), jnp.float32),
        mesh=mesh,
        scratch_types=[
            pltpu.VMEM((NBUF, CHUNK), jnp.int32),
            pltpu.VMEM((NBUF, CHUNK), jnp.int32),
            pltpu.VMEM((NBUF, CHUNK), jnp.int32),
            pltpu.VMEM((NBUF, CHUNK), jnp.int32),
            pltpu.VMEM((NBUF, CHUNK, EMB), jnp.float32),
            pltpu.SemaphoreType.DMA((NBUF,)),
            pltpu.SemaphoreType.DMA((NBUF,)),
            pltpu.SemaphoreType.DMA((NBUF,)),
        ],
    )
    return fn(a0, a1, a2, table)


def kernel(edge_attr, W0, W1, W2):
    table = _build_table(W0, W1, W2)
    ea = jnp.asarray(edge_attr, jnp.int32)
    return _sc_lookup(ea[:, 0], ea[:, 1], ea[:, 2], table)


# speedup vs baseline: 17.0530x; 17.0530x over previous
import functools

import jax
import jax.numpy as jnp
from jax import lax
from jax.experimental import pallas as pl
from jax.experimental.pallas import tpu as pltpu
from jax.experimental.pallas import tpu_sc as plsc

F0, F1, F2 = 5, 6, 2
EMB = 128
E = 320000
NROWS = F0 * F1 * F2

NC, NS = 2, 16
NW = NC * NS
PER_W = E // NW
CHUNK = 80
NCH = PER_W // CHUNK
NBUF = 5


def _table_body(w0_ref, w1_ref, w2_ref, c_ref):
    r = lax.broadcasted_iota(jnp.int32, (NROWS, 1), 0)
    i0 = r // (F1 * F2)
    i1 = (r // F2) % F1
    i2 = r % F2
    oh0 = (i0 == lax.broadcasted_iota(jnp.int32, (NROWS, F0), 1)).astype(jnp.float32)
    oh1 = (i1 == lax.broadcasted_iota(jnp.int32, (NROWS, F1), 1)).astype(jnp.float32)
    oh2 = (i2 == lax.broadcasted_iota(jnp.int32, (NROWS, F2), 1)).astype(jnp.float32)
    acc = jnp.dot(oh0, w0_ref[...], preferred_element_type=jnp.float32)
    acc = acc + jnp.dot(oh1, w1_ref[...], preferred_element_type=jnp.float32)
    acc = acc + jnp.dot(oh2, w2_ref[...], preferred_element_type=jnp.float32)
    c_ref[...] = acc


def _build_table(w0, w1, w2):
    return pl.pallas_call(
        _table_body,
        out_shape=jax.ShapeDtypeStruct((NROWS, EMB), jnp.float32),
    )(w0, w1, w2)


def _sc_body(a0_hbm, a1_hbm, a2_hbm, c_hbm, out_hbm,
             a0_v, a1_v, a2_v, idx_v, rows_v, c_sh,
             isem, gsem, osem):
    sid = lax.axis_index("s")
    wid = sid * NC + lax.axis_index("c")
    wbase = wid * PER_W

    @pl.when(sid == 0)
    def _():
        pltpu.sync_copy(c_hbm, c_sh)
    plsc.subcore_barrier()

    def fire_in(k, b):
        base = wbase + k * CHUNK
        pltpu.async_copy(a0_hbm.at[pl.ds(base, CHUNK)], a0_v.at[b], isem.at[b])
        pltpu.async_copy(a1_hbm.at[pl.ds(base, CHUNK)], a1_v.at[b], isem.at[b])
        pltpu.async_copy(a2_hbm.at[pl.ds(base, CHUNK)], a2_v.at[b], isem.at[b])

    def wait_in(k, b):
        base = wbase + k * CHUNK
        pltpu.make_async_copy(a0_hbm.at[pl.ds(base, CHUNK)], a0_v.at[b], isem.at[b]).wait()
        pltpu.make_async_copy(a1_hbm.at[pl.ds(base, CHUNK)], a1_v.at[b], isem.at[b]).wait()
        pltpu.make_async_copy(a2_hbm.at[pl.ds(base, CHUNK)], a2_v.at[b], isem.at[b]).wait()

    def wait_out(k, b):
        base = wbase + k * CHUNK
        pltpu.make_async_copy(rows_v.at[b], out_hbm.at[pl.ds(base, CHUNK)],
                              osem.at[b]).wait()

    def step(k, b, fire_next, do_wait_out):
        wait_in(k, b)
        for i in range(CHUNK // 16):
            s = pl.ds(i * 16, 16)
            idx_v[b, s] = a0_v[b, s] * (F1 * F2) + a1_v[b, s] * F2 + a2_v[b, s]
        if fire_next:
            fire_in(k + NBUF, b)
        if do_wait_out:
            wait_out(k - NBUF, b)
        pltpu.async_copy(c_sh.at[idx_v.at[b]], rows_v.at[b], gsem.at[b]).wait()
        base = wbase + k * CHUNK
        pltpu.async_copy(rows_v.at[b], out_hbm.at[pl.ds(base, CHUNK)], osem.at[b])

    for b in range(NBUF):
        fire_in(b, b)
    for b in range(NBUF):
        step(b, b, fire_next=True, do_wait_out=False)

    def super_step(g, carry):
        for b in range(NBUF):
            step(g * NBUF + b, b, fire_next=True, do_wait_out=True)
        return carry

    lax.fori_loop(1, NCH // NBUF - 1, super_step, 0)
    for b in range(NBUF):
        step((NCH - NBUF) + b, b, fire_next=False, do_wait_out=True)
    for b in range(NBUF):
        wait_out((NCH - NBUF) + b, b)


@functools.partial(jax.jit, static_argnames=())
def _sc_lookup(a0, a1, a2, table):
    mesh = plsc.VectorSubcoreMesh(core_axis_name="c", subcore_axis_name="s")
    fn = pl.kernel(
        _sc_body,
        out_type=jax.ShapeDtypeStruct((E, EMB), jnp.float32),
        mesh=mesh,
        scratch_types=[
            pltpu.VMEM((NBUF, CHUNK), jnp.int32),
            pltpu.VMEM((NBUF, CHUNK), jnp.int32),
            pltpu.VMEM((NBUF, CHUNK), jnp.int32),
            pltpu.VMEM((NBUF, CHUNK), jnp.int32),
            pltpu.VMEM((NBUF, CHUNK, EMB), jnp.float32),
            pltpu.VMEM_SHARED((NROWS, EMB), jnp.float32),
            pltpu.SemaphoreType.DMA((NBUF,)),
            pltpu.SemaphoreType.DMA((NBUF,)),
            pltpu.SemaphoreType.DMA((NBUF,)),
        ],
    )
    return fn(a0, a1, a2, table)


def kernel(edge_attr, W0, W1, W2):
    table = _build_table(W0, W1, W2)
    ea = jnp.asarray(edge_attr, jnp.int32)
    return _sc_lookup(ea[:, 0], ea[:, 1], ea[:, 2], table)
